# baseline (speedup 1.0000x reference)
import jax
import jax.numpy as jnp
from jax import lax
from jax.experimental import pallas as pl
from jax.experimental.pallas import tpu as pltpu

_N = 1024
_D = 512
_BIG = 1 << 30

_SCAN_ROWS = 512
_SCAN_STEPS = _N // _SCAN_ROWS
_OUT_ROWS = 512
_OUT_STEPS = _N // _OUT_ROWS
_GRID = _SCAN_STEPS + _OUT_STEPS


def _body(b1_ref, b2_ref, ba1_ref, ba2_ref,
          adj_any, nf_any, w1_any, w2_any, wa1_any, wa2_any,
          adj_ref, nf_ref, out_ref,
          mxs, addv, ablk, w1s, w2s, wa1s, wa2s, xs, sem):
    step = pl.program_id(0)

    @pl.when(step < _SCAN_STEPS)
    def _():
        bm = jnp.max(adj_ref[...])
        prev = jnp.where(step == 0, jnp.float32(-3.0e38), mxs[0])
        mxs[0] = jnp.maximum(prev, bm)

    @pl.when(step == _SCAN_STEPS - 1)
    def _():
        found = mxs[0] > 0.0

        @pl.when(jnp.logical_not(found))
        def _():
            addv[...] = jnp.zeros((1, _D), jnp.float32)

        @pl.when(found)
        def _():
            rows = _N // 8

            def blk(i, carry):
                cnt, b1, b2 = carry
                cp = pltpu.make_async_copy(
                    adj_any.at[pl.ds(i * rows, rows), :], ablk, sem)
                cp.start()
                cp.wait()
                a = ablk[...]
                m = a > 0.0
                cnt = cnt + jnp.sum(m.astype(jnp.int32))
                pos = (i * (rows * _N)
                       + lax.broadcasted_iota(jnp.int32, (rows, _N), 0) * _N
                       + lax.broadcasted_iota(jnp.int32, (rows, _N), 1))
                p = jnp.where(m, pos, _BIG)
                p0 = jnp.min(p)
                p1 = jnp.min(jnp.where(p == p0, _BIG, p))
                nb1 = jnp.minimum(b1, p0)
                nb2 = jnp.minimum(jnp.maximum(b1, p0), jnp.minimum(b2, p1))
                return cnt, nb1, nb2

            cnt, f1, f2 = lax.fori_loop(
                0, 8, blk, (jnp.int32(0), jnp.int32(_BIG), jnp.int32(_BIG)))

            idx0 = jnp.where(cnt >= 1, f1, 0)
            idx1 = jnp.where(cnt >= 2, f2, 0)
            src0 = idx0 // _N
            dst0 = idx0 % _N
            src1 = idx1 // _N
            dst1 = idx1 % _N

            copies = [
                pltpu.make_async_copy(src_any, dst, sem)
                for src_any, dst in ((w1_any, w1s), (w2_any, w2s),
                                     (wa1_any, wa1s), (wa2_any, wa2s))
            ]
            copies.extend(
                pltpu.make_async_copy(
                    nf_any.at[pl.ds(row, 1), :],
                    xs.at[pl.ds(r, 1), pl.ds(c, _D)], sem)
                for row, (r, c) in ((src0, (0, 0)), (dst0, (0, _D)),
                                    (src1, (1, 0)), (dst1, (1, _D))))
            for cp in copies:
                cp.start()
            for cp in copies:
                cp.wait()

            hp = lax.dot_general(
                xs[...], w1s[...], (((1,), (0,)), ((), ())),
                preferred_element_type=jnp.float32) + b1_ref[...]
            stepf = lax.dot_general(
                jnp.maximum(hp, 0.0), w2s[...], (((1,), (0,)), ((), ())),
                preferred_element_type=jnp.float32) + b2_ref[...]
            h0 = lax.dot_general(
                stepf[0:1, :], wa1s[0:_D, :], (((1,), (0,)), ((), ())),
                preferred_element_type=jnp.float32)
            h1 = lax.dot_general(
                stepf[1:2, :], wa1s[_D:2 * _D, :], (((1,), (0,)), ((), ())),
                preferred_element_type=jnp.float32)
            h = jnp.maximum(h0 + h1 + ba1_ref[...], 0.0)
            addv[...] = lax.dot_general(
                h, wa2s[...], (((1,), (0,)), ((), ())),
                preferred_element_type=jnp.float32) + ba2_ref[...]

    @pl.when(step >= _SCAN_STEPS)
    def _():
        out_ref[...] = nf_ref[...] + addv[...]


def kernel(node_features, adjacency_matrix, edge_types, W1, b1, W2, b2,
           Ws1, bs1, Ws2, bs2, Wa1, ba1, Wa2, ba2):
    del edge_types, Ws1, bs1, Ws2, bs2
    bias = pl.BlockSpec((1, _D), lambda g: (0, 0))
    hbm = pl.BlockSpec(memory_space=pltpu.MemorySpace.HBM)
    return pl.pallas_call(
        _body,
        grid=(_GRID,),
        in_specs=[bias, bias, bias, bias,
                  hbm, hbm, hbm, hbm, hbm, hbm,
                  pl.BlockSpec((_SCAN_ROWS, _N),
                               lambda g: (jnp.minimum(g, _SCAN_STEPS - 1), 0)),
                  pl.BlockSpec((_OUT_ROWS, _D),
                               lambda g: (jnp.maximum(g - _SCAN_STEPS, 0), 0))],
        out_specs=pl.BlockSpec((_OUT_ROWS, _D),
                               lambda g: (jnp.maximum(g - _SCAN_STEPS, 0), 0)),
        out_shape=jax.ShapeDtypeStruct((_N, _D), jnp.float32),
        scratch_shapes=[
            pltpu.SMEM((1,), jnp.float32),
            pltpu.VMEM((1, _D), jnp.float32),
            pltpu.VMEM((_N // 8, _N), jnp.float32),
            pltpu.VMEM((2 * _D, _D), jnp.float32),
            pltpu.VMEM((_D, _D), jnp.float32),
            pltpu.VMEM((2 * _D, _D), jnp.float32),
            pltpu.VMEM((_D, _D), jnp.float32),
            pltpu.VMEM((2, 2 * _D), jnp.float32),
            pltpu.SemaphoreType.DMA,
        ],
    )(b1.reshape(1, _D), b2.reshape(1, _D), ba1.reshape(1, _D),
      ba2.reshape(1, _D), adjacency_matrix, node_features, W1, W2, Wa1, Wa2,
      adjacency_matrix, node_features)


# speedup vs baseline: 1.1101x; 1.1101x over previous
import jax
import jax.numpy as jnp
from jax import lax
from jax.experimental import pallas as pl
from jax.experimental.pallas import tpu as pltpu

_N = 1024
_D = 512
_BIG = 1 << 30

_ROWS = 256
_STEPS = _N // _ROWS


def _body(b1_ref, b2_ref, ba1_ref, ba2_ref,
          adj_any, nf_any, w1_any, w2_any, wa1_any, wa2_any,
          adj_ref, nf_ref, out_any,
          mxs, addv, hold, ablk, w1s, w2s, wa1s, wa2s, xs, sem, osem):
    step = pl.program_id(0)

    bm = jnp.max(adj_ref[...])
    prev = jnp.where(step == 0, jnp.float32(-3.0e38), mxs[0])
    mxs[0] = jnp.maximum(prev, bm)
    hold[pl.ds(step * _ROWS, _ROWS), :] = nf_ref[...]

    @pl.when(step == _STEPS - 1)
    def _():
        found = mxs[0] > 0.0

        @pl.when(jnp.logical_not(found))
        def _():
            addv[...] = jnp.zeros((1, _D), jnp.float32)

        @pl.when(found)
        def _():
            rows = _N // 8

            def blk(i, carry):
                cnt, b1, b2 = carry
                cp = pltpu.make_async_copy(
                    adj_any.at[pl.ds(i * rows, rows), :], ablk, sem)
                cp.start()
                cp.wait()
                a = ablk[...]
                m = a > 0.0
                cnt = cnt + jnp.sum(m.astype(jnp.int32))
                pos = (i * (rows * _N)
                       + lax.broadcasted_iota(jnp.int32, (rows, _N), 0) * _N
                       + lax.broadcasted_iota(jnp.int32, (rows, _N), 1))
                p = jnp.where(m, pos, _BIG)
                p0 = jnp.min(p)
                p1 = jnp.min(jnp.where(p == p0, _BIG, p))
                nb1 = jnp.minimum(b1, p0)
                nb2 = jnp.minimum(jnp.maximum(b1, p0), jnp.minimum(b2, p1))
                return cnt, nb1, nb2

            cnt, f1, f2 = lax.fori_loop(
                0, 8, blk, (jnp.int32(0), jnp.int32(_BIG), jnp.int32(_BIG)))

            idx0 = jnp.where(cnt >= 1, f1, 0)
            idx1 = jnp.where(cnt >= 2, f2, 0)
            src0 = idx0 // _N
            dst0 = idx0 % _N
            src1 = idx1 // _N
            dst1 = idx1 % _N

            copies = [
                pltpu.make_async_copy(src_any, dst, sem)
                for src_any, dst in ((w1_any, w1s), (w2_any, w2s),
                                     (wa1_any, wa1s), (wa2_any, wa2s))
            ]
            copies.extend(
                pltpu.make_async_copy(
                    nf_any.at[pl.ds(row, 1), :],
                    xs.at[pl.ds(r, 1), pl.ds(c, _D)], sem)
                for row, (r, c) in ((src0, (0, 0)), (dst0, (0, _D)),
                                    (src1, (1, 0)), (dst1, (1, _D))))
            for cp in copies:
                cp.start()
            for cp in copies:
                cp.wait()

            hp = lax.dot_general(
                xs[...], w1s[...], (((1,), (0,)), ((), ())),
                preferred_element_type=jnp.float32) + b1_ref[...]
            stepf = lax.dot_general(
                jnp.maximum(hp, 0.0), w2s[...], (((1,), (0,)), ((), ())),
                preferred_element_type=jnp.float32) + b2_ref[...]
            h0 = lax.dot_general(
                stepf[0:1, :], wa1s[0:_D, :], (((1,), (0,)), ((), ())),
                preferred_element_type=jnp.float32)
            h1 = lax.dot_general(
                stepf[1:2, :], wa1s[_D:2 * _D, :], (((1,), (0,)), ((), ())),
                preferred_element_type=jnp.float32)
            h = jnp.maximum(h0 + h1 + ba1_ref[...], 0.0)
            addv[...] = lax.dot_general(
                h, wa2s[...], (((1,), (0,)), ((), ())),
                preferred_element_type=jnp.float32) + ba2_ref[...]

        ocopies = []
        for i in range(_STEPS):
            sl = pl.ds(i * _ROWS, _ROWS)
            hold[sl, :] = hold[sl, :] + addv[...]
            cp = pltpu.make_async_copy(
                hold.at[sl, :], out_any.at[sl, :], osem)
            cp.start()
            ocopies.append(cp)
        for cp in ocopies:
            cp.wait()


def kernel(node_features, adjacency_matrix, edge_types, W1, b1, W2, b2,
           Ws1, bs1, Ws2, bs2, Wa1, ba1, Wa2, ba2):
    del edge_types, Ws1, bs1, Ws2, bs2
    bias = pl.BlockSpec((1, _D), lambda g: (0, 0))
    hbm = pl.BlockSpec(memory_space=pltpu.MemorySpace.HBM)
    return pl.pallas_call(
        _body,
        grid=(_STEPS,),
        in_specs=[bias, bias, bias, bias,
                  hbm, hbm, hbm, hbm, hbm, hbm,
                  pl.BlockSpec((_ROWS, _N), lambda g: (g, 0)),
                  pl.BlockSpec((_ROWS, _D), lambda g: (g, 0))],
        out_specs=hbm,
        out_shape=jax.ShapeDtypeStruct((_N, _D), jnp.float32),
        scratch_shapes=[
            pltpu.SMEM((1,), jnp.float32),
            pltpu.VMEM((1, _D), jnp.float32),
            pltpu.VMEM((_N, _D), jnp.float32),
            pltpu.VMEM((_N // 8, _N), jnp.float32),
            pltpu.VMEM((2 * _D, _D), jnp.float32),
            pltpu.VMEM((_D, _D), jnp.float32),
            pltpu.VMEM((2 * _D, _D), jnp.float32),
            pltpu.VMEM((_D, _D), jnp.float32),
            pltpu.VMEM((2, 2 * _D), jnp.float32),
            pltpu.SemaphoreType.DMA,
            pltpu.SemaphoreType.DMA,
        ],
    )(b1.reshape(1, _D), b2.reshape(1, _D), ba1.reshape(1, _D),
      ba2.reshape(1, _D), adjacency_matrix, node_features, W1, W2, Wa1, Wa2,
      adjacency_matrix, node_features)
